# baseline (speedup 1.0000x reference)
import functools

import jax
import jax.numpy as jnp
from jax import lax
from jax.experimental import pallas as pl
from jax.experimental.pallas import tpu as pltpu
from jax.experimental.pallas import tpu_sc as plsc

NUM_FIELDS = 26
VOCAB = 100000
EMBED_DIM = 32
BATCH = 16384

N_ROWS = BATCH * NUM_FIELDS
NC, NS = 2, 16
NW = NC * NS
ROWS_PER_W = N_ROWS // NW
K = 128
G = 8
CHUNKS_PER_W = ROWS_PER_W // K
GROUPS = CHUNKS_PER_W // G

_mesh = plsc.VectorSubcoreMesh(core_axis_name="c", subcore_axis_name="s")


@functools.partial(
    pl.kernel,
    out_type=jax.ShapeDtypeStruct((N_ROWS, EMBED_DIM), jnp.float32),
    mesh=_mesh,
    scratch_types=[
        pltpu.VMEM((CHUNKS_PER_W, K), jnp.int32),
        pltpu.VMEM((G * K, EMBED_DIM), jnp.float32),
        pltpu.SemaphoreType.DMA,
    ],
    compiler_params=pltpu.CompilerParams(use_tc_tiling_on_sc=False),
)
def _gather_kernel(table_hbm, idx_hbm, out_hbm, idx_v, rows_v, sem):
    wid = lax.axis_index("s") * NC + lax.axis_index("c")
    pltpu.sync_copy(idx_hbm.at[pl.ds(wid * CHUNKS_PER_W, CHUNKS_PER_W)], idx_v)
    base = wid * ROWS_PER_W

    def group(g, carry):
        copies = [
            pltpu.async_copy(
                table_hbm.at[idx_v.at[g * G + j]],
                rows_v.at[pl.ds(j * K, K)],
                sem,
            )
            for j in range(G)
        ]
        for c in copies:
            c.wait()
        pltpu.sync_copy(rows_v, out_hbm.at[pl.ds(base + g * (G * K), G * K)])
        return carry

    lax.fori_loop(0, GROUPS, group, 0)


def kernel(inputs, tables):
    offsets = (jnp.arange(NUM_FIELDS, dtype=jnp.int32) * VOCAB)[None, :]
    ii = inputs.astype(jnp.int32)
    packed = ii.T * 16384 + jnp.arange(BATCH, dtype=jnp.int32)[None, :]
    srt = jnp.sort(packed, axis=-1)
    ii = ii + jnp.minimum(srt[0, 0], 0)[None, None]
    gidx = (ii + offsets).reshape(NW * CHUNKS_PER_W, K)
    flat_table = tables.reshape(NUM_FIELDS * VOCAB, EMBED_DIM)
    out = _gather_kernel(flat_table, gidx)
    return out.reshape(BATCH, NUM_FIELDS * EMBED_DIM)


# speedup vs baseline: 1.4536x; 1.4536x over previous
import functools

import jax
import jax.numpy as jnp
from jax import lax
from jax.experimental import pallas as pl
from jax.experimental.pallas import tpu as pltpu
from jax.experimental.pallas import tpu_sc as plsc

NUM_FIELDS = 26
VOCAB = 100000
EMBED_DIM = 32
BATCH = 16384

N_ROWS = BATCH * NUM_FIELDS
NC, NS = 2, 16
NW = NC * NS
ROWS_PER_W = N_ROWS // NW
VREGS_PER_W = ROWS_PER_W // 16
FLUSH_VREGS = 32
FLUSHES = VREGS_PER_W // FLUSH_VREGS
WIN_T = 1024
TAIL_S = (VOCAB - 1) // WIN_T * WIN_T
TAIL_W = VOCAB - TAIL_S
IROWS_PER_W = ROWS_PER_W // 4
CHUNK = 1024
CHUNKS2 = ROWS_PER_W // CHUNK

_mesh = plsc.VectorSubcoreMesh(core_axis_name="c", subcore_axis_name="s")


@functools.partial(
    pl.kernel,
    out_type=jax.ShapeDtypeStruct((N_ROWS // 4, 128), jnp.float32),
    mesh=_mesh,
    scratch_types=[
        pltpu.VMEM((ROWS_PER_W,), jnp.int32),
        pltpu.VMEM((32, WIN_T), jnp.float32),
        pltpu.VMEM((256, 128), jnp.float32),
        pltpu.SemaphoreType.DMA,
    ],
    compiler_params=pltpu.CompilerParams(
        use_tc_tiling_on_sc=True, needs_layout_passes=False
    ),
)
def _extract_kernel(tab_hbm, tail_hbm, srt_hbm, inter_hbm, items_v, wbuf, obuf, osem):
    wid = lax.axis_index("s") * NC + lax.axis_index("c")
    base = wid * ROWS_PER_W
    pltpu.sync_copy(srt_hbm.at[pl.ds(base, ROWS_PER_W)], items_v)

    iota = lax.iota(jnp.int32, 16)
    zeros16 = jnp.zeros((16,), jnp.int32)

    def flush_group(f, st):
        fpar = lax.rem(f, 2)

        @pl.when(f >= 2)
        def _():
            pltpu.make_async_copy(
                inter_hbm.at[pl.ds(0, 128)], obuf.at[pl.ds(0, 128)], osem
            ).wait()

        def vreg(v2, st2):
            v = f * FLUSH_VREGS + v2
            lanes = v * 16 + iota
            pk = plsc.load_gather(items_v, [lanes])
            fld = lax.div(base + v * 16, BATCH)
            t = lax.shift_right_logical(pk, 14)
            slot = v2 * 16 + iota
            orow = fpar * 128 + lax.shift_right_logical(slot, 2)
            ocol0 = lax.bitwise_and(slot, 3) * EMBED_DIM

            def wcond(st3):
                m, _, _ = st3
                return jnp.any(m)

            def wbody(st3):
                m, s, fl = st3
                tmin = jnp.min(jnp.where(m, t, 1 << 29))
                need = (fl != fld) | (tmin >= s + WIN_T) | (tmin < s)
                new_s = jnp.where(
                    need,
                    lax.shift_left(lax.shift_right_logical(tmin, 10), 10),
                    s,
                )

                @pl.when(need & (new_s != TAIL_S))
                def _():
                    pltpu.sync_copy(
                        tab_hbm.at[fld, :, pl.ds(pl.multiple_of(new_s, 1024), WIN_T)],
                        wbuf
                    )

                @pl.when(need & (new_s == TAIL_S))
                def _():
                    pltpu.sync_copy(
                        tail_hbm.at[fld],
                        wbuf.at[:, pl.ds(0, 768)],
                    )

                inwin = m & (t >= new_s) & (t < new_s + WIN_T)
                tloc = t - new_s
                for d in range(EMBED_DIM):
                    vals = plsc.load_gather(
                        wbuf, [zeros16 + d, tloc], mask=inwin
                    )
                    plsc.store_scatter(
                        obuf, [orow, ocol0 + d], vals, mask=inwin
                    )
                return (m & ~inwin, new_s, jnp.where(need, fld, fl))

            m0 = iota < 16
            _, s_out, fl_out = lax.while_loop(wcond, wbody, (m0,) + st2)
            return (s_out, fl_out)

        st = lax.fori_loop(0, FLUSH_VREGS, vreg, st)
        pltpu.async_copy(
            obuf.at[pl.ds(fpar * 128, 128)],
            inter_hbm.at[pl.ds(wid * IROWS_PER_W + f * 128, 128)],
            osem,
        )
        return st

    lax.fori_loop(0, FLUSHES, flush_group,
                  (jnp.full((), 0, jnp.int32), jnp.full((), -1, jnp.int32)))

    for _ in range(2):
        pltpu.make_async_copy(
            inter_hbm.at[pl.ds(0, 128)], obuf.at[pl.ds(0, 128)], osem
        ).wait()


@functools.partial(
    pl.kernel,
    out_type=jax.ShapeDtypeStruct((N_ROWS, EMBED_DIM), jnp.float32),
    mesh=_mesh,
    scratch_types=[
        pltpu.VMEM((2 * CHUNK,), jnp.int32),
        pltpu.VMEM((2 * CHUNK, EMBED_DIM), jnp.float32),
        pltpu.VMEM((16, 128), jnp.int32),
        pltpu.SemaphoreType.DMA,
        pltpu.SemaphoreType.DMA,
    ],
    compiler_params=pltpu.CompilerParams(
        use_tc_tiling_on_sc=False, needs_layout_passes=False
    ),
)
def _scatter_kernel(inter_hbm, srt_hbm, out_hbm, pk_v, rbuf, rid_v, rsem, osem):
    wid = lax.axis_index("s") * NC + lax.axis_index("c")
    base = wid * ROWS_PER_W

    iota = lax.iota(jnp.int32, 16)

    def issue_reads(c, par):
        pltpu.async_copy(
            srt_hbm.at[pl.ds(base + c * CHUNK, CHUNK)],
            pk_v.at[pl.ds(par * CHUNK, CHUNK)],
            rsem,
        )
        pltpu.async_copy(
            inter_hbm.at[pl.ds(base + c * CHUNK, CHUNK)],
            rbuf.at[pl.ds(par * CHUNK, CHUNK)],
            rsem,
        )

    issue_reads(0, 0)

    def chunk(c, carry):
        par = lax.rem(c, 2)

        pltpu.make_async_copy(
            srt_hbm.at[pl.ds(0, CHUNK)], pk_v.at[pl.ds(0, CHUNK)], rsem
        ).wait()
        pltpu.make_async_copy(
            inter_hbm.at[pl.ds(0, CHUNK)], rbuf.at[pl.ds(0, CHUNK)], rsem
        ).wait()

        @pl.when(c >= 2)
        def _():
            for _ in range(8):
                pltpu.make_async_copy(
                    inter_hbm.at[pl.ds(0, 128)],
                    rbuf.at[pl.ds(0, 128)],
                    osem,
                ).wait()

        @pl.when(c + 1 < CHUNKS2)
        def _():
            issue_reads(c + 1, 1 - par)

        fld = lax.div(base + c * CHUNK, BATCH)

        def rvreg(v, carry2):
            off = par * CHUNK + v * 16 + iota
            pkv = plsc.load_gather(pk_v, [off])
            rid = lax.bitwise_and(pkv, BATCH - 1) * NUM_FIELDS + fld
            row = par * 8 + lax.div(v, 8)
            col = lax.rem(v, 8) * 16 + iota
            plsc.store_scatter(
                rid_v, [row + jnp.zeros((16,), jnp.int32), col], rid
            )
            return carry2

        lax.fori_loop(0, CHUNK // 16, rvreg, 0)

        for j in range(8):
            pltpu.async_copy(
                rbuf.at[pl.ds(par * CHUNK + j * 128, 128)],
                out_hbm.at[rid_v.at[par * 8 + j]],
                osem,
            )
        return carry

    lax.fori_loop(0, CHUNKS2, chunk, 0)

    for _ in range(16):
        pltpu.make_async_copy(
            inter_hbm.at[pl.ds(0, 128)], rbuf.at[pl.ds(0, 128)], osem
        ).wait()


def kernel(inputs, tables):
    ii = inputs.astype(jnp.int32)
    packed = ii.T * BATCH + jnp.arange(BATCH, dtype=jnp.int32)[None, :]
    srt = jnp.sort(packed, axis=-1).reshape(N_ROWS)
    tab_t = jnp.transpose(tables, (0, 2, 1))
    tail_t = jnp.pad(
        tab_t[:, :, TAIL_S:], ((0, 0), (0, 0), (0, 768 - TAIL_W))
    )
    inter = _extract_kernel(tab_t, tail_t, srt)
    inter2 = inter.reshape(N_ROWS, EMBED_DIM)
    out = _scatter_kernel(inter2, srt)
    return out.reshape(BATCH, NUM_FIELDS * EMBED_DIM)


# speedup vs baseline: 1.4955x; 1.0288x over previous
import functools

import jax
import jax.numpy as jnp
from jax import lax
from jax.experimental import pallas as pl
from jax.experimental.pallas import tpu as pltpu
from jax.experimental.pallas import tpu_sc as plsc

NUM_FIELDS = 26
VOCAB = 100000
EMBED_DIM = 32
BATCH = 16384

N_ROWS = BATCH * NUM_FIELDS
NC, NS = 2, 16
NW = NC * NS
ROWS_PER_W = N_ROWS // NW
VREGS_PER_W = ROWS_PER_W // 16
FLUSH_VREGS = 32
FLUSHES = VREGS_PER_W // FLUSH_VREGS
WIN_T = 2048
TAIL_S = (VOCAB - 1) // WIN_T * WIN_T
TAIL_W = VOCAB - TAIL_S
TAIL_PAD = 1792
IROWS_PER_W = ROWS_PER_W // 4
CHUNK = 1024
CHUNKS2 = ROWS_PER_W // CHUNK

_mesh = plsc.VectorSubcoreMesh(core_axis_name="c", subcore_axis_name="s")


@functools.partial(
    pl.kernel,
    out_type=jax.ShapeDtypeStruct((N_ROWS // 4, 128), jnp.float32),
    mesh=_mesh,
    scratch_types=[
        pltpu.VMEM((ROWS_PER_W,), jnp.int32),
        pltpu.VMEM((32, WIN_T), jnp.float32),
        pltpu.VMEM((256, 128), jnp.float32),
        pltpu.SemaphoreType.DMA,
    ],
    compiler_params=pltpu.CompilerParams(
        use_tc_tiling_on_sc=True, needs_layout_passes=False
    ),
)
def _extract_kernel(tab_hbm, tail_hbm, srt_hbm, inter_hbm, items_v, wbuf, obuf, osem):
    wid = lax.axis_index("s") * NC + lax.axis_index("c")
    base = wid * ROWS_PER_W
    pltpu.sync_copy(srt_hbm.at[pl.ds(base, ROWS_PER_W)], items_v)

    iota = lax.iota(jnp.int32, 16)
    zeros16 = jnp.zeros((16,), jnp.int32)

    def flush_group(f, st):
        fpar = lax.rem(f, 2)

        @pl.when(f >= 2)
        def _():
            pltpu.make_async_copy(
                inter_hbm.at[pl.ds(0, 128)], obuf.at[pl.ds(0, 128)], osem
            ).wait()

        def vreg(v2, st2):
            v = f * FLUSH_VREGS + v2
            lanes = v * 16 + iota
            pk = plsc.load_gather(items_v, [lanes])
            fld = lax.div(base + v * 16, BATCH)
            t = lax.shift_right_logical(pk, 14)
            slot = v2 * 16 + iota
            orow = fpar * 128 + lax.shift_right_logical(slot, 2)
            ocol0 = lax.bitwise_and(slot, 3) * EMBED_DIM

            def wcond(st3):
                m, _, _ = st3
                return jnp.any(m)

            def wbody(st3):
                m, s, fl = st3
                tmin = jnp.min(jnp.where(m, t, 1 << 29))
                need = (fl != fld) | (tmin >= s + WIN_T) | (tmin < s)
                new_s = jnp.where(
                    need,
                    lax.shift_left(lax.shift_right_logical(tmin, 11), 11),
                    s,
                )

                @pl.when(need & (new_s != TAIL_S))
                def _():
                    pltpu.sync_copy(
                        tab_hbm.at[fld, :, pl.ds(pl.multiple_of(new_s, 2048), WIN_T)],
                        wbuf
                    )

                @pl.when(need & (new_s == TAIL_S))
                def _():
                    pltpu.sync_copy(
                        tail_hbm.at[fld],
                        wbuf.at[:, pl.ds(0, TAIL_PAD)],
                    )

                inwin = m & (t >= new_s) & (t < new_s + WIN_T)
                tloc = t - new_s
                for d in range(EMBED_DIM):
                    vals = plsc.load_gather(
                        wbuf, [zeros16 + d, tloc], mask=inwin
                    )
                    plsc.store_scatter(
                        obuf, [orow, ocol0 + d], vals, mask=inwin
                    )
                return (m & ~inwin, new_s, jnp.where(need, fld, fl))

            m0 = iota < 16
            _, s_out, fl_out = lax.while_loop(wcond, wbody, (m0,) + st2)
            return (s_out, fl_out)

        st = lax.fori_loop(0, FLUSH_VREGS, vreg, st)
        pltpu.async_copy(
            obuf.at[pl.ds(fpar * 128, 128)],
            inter_hbm.at[pl.ds(wid * IROWS_PER_W + f * 128, 128)],
            osem,
        )
        return st

    lax.fori_loop(0, FLUSHES, flush_group,
                  (jnp.full((), 0, jnp.int32), jnp.full((), -1, jnp.int32)))

    for _ in range(2):
        pltpu.make_async_copy(
            inter_hbm.at[pl.ds(0, 128)], obuf.at[pl.ds(0, 128)], osem
        ).wait()


@functools.partial(
    pl.kernel,
    out_type=jax.ShapeDtypeStruct((N_ROWS, EMBED_DIM), jnp.float32),
    mesh=_mesh,
    scratch_types=[
        pltpu.VMEM((2 * CHUNK,), jnp.int32),
        pltpu.VMEM((2 * CHUNK, EMBED_DIM), jnp.float32),
        pltpu.VMEM((16, 128), jnp.int32),
        pltpu.SemaphoreType.DMA,
        pltpu.SemaphoreType.DMA,
    ],
    compiler_params=pltpu.CompilerParams(
        use_tc_tiling_on_sc=False, needs_layout_passes=False
    ),
)
def _scatter_kernel(inter_hbm, srt_hbm, out_hbm, pk_v, rbuf, rid_v, rsem, osem):
    wid = lax.axis_index("s") * NC + lax.axis_index("c")
    base = wid * ROWS_PER_W

    iota = lax.iota(jnp.int32, 16)

    def issue_reads(c, par):
        pltpu.async_copy(
            srt_hbm.at[pl.ds(base + c * CHUNK, CHUNK)],
            pk_v.at[pl.ds(par * CHUNK, CHUNK)],
            rsem,
        )
        pltpu.async_copy(
            inter_hbm.at[pl.ds(base + c * CHUNK, CHUNK)],
            rbuf.at[pl.ds(par * CHUNK, CHUNK)],
            rsem,
        )

    issue_reads(0, 0)

    def chunk(c, carry):
        par = lax.rem(c, 2)

        pltpu.make_async_copy(
            srt_hbm.at[pl.ds(0, CHUNK)], pk_v.at[pl.ds(0, CHUNK)], rsem
        ).wait()
        pltpu.make_async_copy(
            inter_hbm.at[pl.ds(0, CHUNK)], rbuf.at[pl.ds(0, CHUNK)], rsem
        ).wait()

        @pl.when(c >= 2)
        def _():
            for _ in range(8):
                pltpu.make_async_copy(
                    inter_hbm.at[pl.ds(0, 128)],
                    rbuf.at[pl.ds(0, 128)],
                    osem,
                ).wait()

        @pl.when(c + 1 < CHUNKS2)
        def _():
            issue_reads(c + 1, 1 - par)

        fld = lax.div(base + c * CHUNK, BATCH)

        def rvreg(v, carry2):
            off = par * CHUNK + v * 16 + iota
            pkv = plsc.load_gather(pk_v, [off])
            rid = lax.bitwise_and(pkv, BATCH - 1) * NUM_FIELDS + fld
            row = par * 8 + lax.div(v, 8)
            col = lax.rem(v, 8) * 16 + iota
            plsc.store_scatter(
                rid_v, [row + jnp.zeros((16,), jnp.int32), col], rid
            )
            return carry2

        lax.fori_loop(0, CHUNK // 16, rvreg, 0)

        for j in range(8):
            pltpu.async_copy(
                rbuf.at[pl.ds(par * CHUNK + j * 128, 128)],
                out_hbm.at[rid_v.at[par * 8 + j]],
                osem,
            )
        return carry

    lax.fori_loop(0, CHUNKS2, chunk, 0)

    for _ in range(16):
        pltpu.make_async_copy(
            inter_hbm.at[pl.ds(0, 128)], rbuf.at[pl.ds(0, 128)], osem
        ).wait()


def kernel(inputs, tables):
    ii = inputs.astype(jnp.int32)
    packed = ii.T * BATCH + jnp.arange(BATCH, dtype=jnp.int32)[None, :]
    srt = jnp.sort(packed, axis=-1).reshape(N_ROWS)
    tab_t = jnp.transpose(tables, (0, 2, 1))
    tail_t = jnp.pad(
        tab_t[:, :, TAIL_S:], ((0, 0), (0, 0), (0, TAIL_PAD - TAIL_W))
    )
    inter = _extract_kernel(tab_t, tail_t, srt)
    inter2 = inter.reshape(N_ROWS, EMBED_DIM)
    out = _scatter_kernel(inter2, srt)
    return out.reshape(BATCH, NUM_FIELDS * EMBED_DIM)
